# baseline (speedup 1.0000x reference)
import functools

import jax
import jax.numpy as jnp
from jax import lax
from jax.experimental import pallas as pl
from jax.experimental.pallas import tpu as pltpu
from jax.experimental.pallas import tpu_sc as plsc

F32 = jnp.float32
HI = jax.lax.Precision.HIGHEST

N = 10000
E = 160000
D = 300
H = D // 2
HP = 160
NUM_EMB = 87
L = 5

NSC = 2
NT = 16
CHUNK = 40
EPT = E // NT
NCH = EPT // CHUNK

NB = 10
BR = N // NB


ZPT = ((N + 1 + NT - 1) // NT + 7) // 8 * 8
ACC_ROWS = ZPT * NT

_B1 = [5040 * k for k in range(1, 9)]
_B2 = [40320 + 4960 * k for k in range(1, 9)]
SHARD_CUTS = (_B1 + _B2 + [80000 + b for b in _B1] + [80000 + b for b in _B2])
SHARD_CUTS = sorted(set(c for c in SHARD_CUTS if 0 < c < E))
SLOTS = 32


def _dot_t(a, b):
    return lax.dot_general(a.astype(jnp.bfloat16), b.astype(jnp.bfloat16),
                           (((1,), (1,)), ((), ())),
                           preferred_element_type=F32)


def _r(a):
    return a.astype(jnp.bfloat16).astype(F32)


def _sc_aggr_body(h_hbm, emb_hbm, src_hbm, eid_hbm, dst_hbm, bnd_hbm,
                  tbl_hbm, out_hbm,
                  emb_v, rows_v, src_v, eid_v, dst_v, bnd_s, tbl_v, acc):
    c = lax.axis_index("c")
    s = lax.axis_index("s")

    @pl.loop(0, ZPT + SLOTS)
    def _(r):
        for k in range(HP // 16):
            acc[pl.ds(r, 1), pl.ds(k * 16, 16)] = jnp.zeros((1, 16), F32)

    pltpu.sync_copy(bnd_hbm.at[s], bnd_s.at[pl.ds(0, 2)])
    bv = bnd_s[pl.ds(0, 16)]
    lo = bv[0]
    hi = bv[1]
    lo8 = (lo // 8) * 8
    nch = (hi - lo8 + CHUNK - 1) // CHUNK
    rbase = s * ZPT

    def chunk(j, carry):
        base = lo8 + j * CHUNK
        pltpu.sync_copy(src_hbm.at[c, pl.ds(base, CHUNK)], src_v)
        pltpu.sync_copy(eid_hbm.at[c, pl.ds(base, CHUNK)], eid_v)
        pltpu.sync_copy(dst_hbm.at[pl.ds(base, CHUNK)],
                        dst_v.at[pl.ds(0, CHUNK)])
        pltpu.sync_copy(h_hbm.at[src_v], rows_v)
        pltpu.sync_copy(emb_hbm.at[eid_v], emb_v)

        r0 = jnp.maximum(lo - base, 0)
        r1 = jnp.minimum(hi - base, CHUNK)

        def row(r, carry2):
            d = dst_v[pl.ds(r, 16)][0] - rbase
            for k in range(HP // 16):
                slc = (pl.ds(d, 1), pl.ds(k * 16, 16))
                mslc = (pl.ds(r, 1), pl.ds(k * 16, 16))
                acc[slc] += jnp.maximum(rows_v[mslc] + emb_v[mslc], 0.0)
            return carry2

        lax.fori_loop(r0, r1, row, 0)
        return carry

    lax.fori_loop(0, nch, chunk, 0)

    pltpu.sync_copy(tbl_hbm.at[s], tbl_v.at[pl.ds(0, SLOTS)])
    for slot in range(SLOTS):
        tgt = tbl_v[pl.ds(slot, 16)][0]
        for k in range(HP // 16):
            slc = (pl.ds(tgt, 1), pl.ds(k * 16, 16))
            acc[slc] += acc[pl.ds(ZPT + slot, 1), pl.ds(k * 16, 16)]

    pltpu.sync_copy(acc.at[pl.ds(0, ZPT)],
                    out_hbm.at[c, pl.ds(s * ZPT, ZPT)])


@functools.cache
def _get_sc_aggr():
    mesh = plsc.VectorSubcoreMesh(core_axis_name="c", subcore_axis_name="s",
                                  num_cores=NSC, num_subcores=NT)
    return pl.kernel(
        _sc_aggr_body,
        out_type=jax.ShapeDtypeStruct((NSC, ACC_ROWS, HP), F32),
        mesh=mesh,
        compiler_params=pltpu.CompilerParams(use_tc_tiling_on_sc=False),
        scratch_types=[
            pltpu.VMEM((CHUNK, HP), F32),
            pltpu.VMEM((CHUNK, HP), F32),
            pltpu.VMEM((CHUNK,), jnp.int32),
            pltpu.VMEM((CHUNK,), jnp.int32),
            pltpu.VMEM((CHUNK + 16,), jnp.int32),
            pltpu.VMEM((16,), jnp.int32),
            pltpu.VMEM((SLOTS + 16,), jnp.int32),
            pltpu.VMEM((ZPT + SLOTS, HP), F32),
        ],
    )


def _split_store(out_ref, x, rows):
    pad = jnp.zeros((rows, HP - H), F32)
    out_ref[0] = jnp.concatenate([x[:, :H], pad], axis=1)
    out_ref[1] = jnp.concatenate([x[:, H:], pad], axis=1)


def _node_prologue_body(z_ref, ch_ref, fc_ref, at_ref, w1t_ref, b1_ref,
                        w2_ref, b2_ref, out_ref):
    zb = z_ref[0]
    oh = (zb == lax.broadcasted_iota(jnp.int32, (BR, NUM_EMB), 1)).astype(F32)
    emb = lax.dot_general(oh, at_ref[...], (((1,), (0,)), ((), ())),
                          preferred_element_type=F32, precision=HI)
    na = jnp.concatenate([ch_ref[0], fc_ref[0]], axis=1)
    pre = _dot_t(na, w1t_ref[...]) + b1_ref[...]
    x = emb + _dot_t(jnp.maximum(pre, 0.0), w2_ref[...]) + b2_ref[...]
    _split_store(out_ref, x, BR)


def _edge_prologue_body(ea_ref, w1t_ref, b1_ref, w2_ref, b2_ref, out_ref):
    pre = _dot_t(ea_ref[...], w1t_ref[...]) + b1_ref[...]
    e = _dot_t(jnp.maximum(pre, 0.0), w2_ref[...]) + b2_ref[...]
    _split_store(out_ref, e, out_ref.shape[1])


def _mlp_body(h_ref, a_ref, w1_ref, b1_ref, w2_ref, b2_ref, t_ref, st_ref):
    i = pl.program_id(0)
    x = jnp.concatenate([h_ref[0][:, :H] + a_ref[0][:, :H],
                         h_ref[1][:, :H] + a_ref[1][:, :H]], axis=1)
    pre = _dot_t(x, w1_ref[...]) + b1_ref[...]
    t = _dot_t(jnp.maximum(pre, 0.0), w2_ref[...]) + b2_ref[...]
    t_ref[...] = t

    @pl.when(i == 0)
    def _():
        st_ref[...] = jnp.zeros_like(st_ref)

    st_ref[0:1, :] += jnp.sum(t, axis=0, keepdims=True)
    st_ref[1:2, :] += jnp.sum(t * t, axis=0, keepdims=True)


def _bn_body(t_ref, st_ref, g_ref, b_ref, out_ref):
    mean = st_ref[0:1, :] * (1.0 / N)
    var = st_ref[1:2, :] * (1.0 / N) - mean * mean
    scale = lax.rsqrt(var + 1e-5) * g_ref[...]
    shift = b_ref[...] - mean * scale
    hnew = jnp.maximum(t_ref[...] * scale + shift, 0.0)
    _split_store(out_ref, hnew, BR)


def _head_body(h_ref, hw_ref, hb_ref, cw_ref, cb_ref, lg_ref, ds_ref):
    x = jnp.concatenate([h_ref[0][:, :H], h_ref[1][:, :H]], axis=1)
    lg_ref[...] = _dot_t(x, hw_ref[...]) + hb_ref[...]
    ds_ref[...] = _dot_t(x, cw_ref[...]) + cb_ref[...]


def _full(r=2):
    return pl.BlockSpec(index_map=lambda i: tuple(0 for _ in range(r)))


def kernel(z, chirality, formal_charge, edge_index, edge_attr, atom_table,
           nap_W1, nap_b1, nap_W2, nap_b2, ee_W1, ee_b1, ee_W2, ee_b2,
           mlp_W1, mlp_b1, mlp_W2, mlp_b2, bn_gamma, bn_beta,
           head_W, head_b, coord_W, coord_b):
    src = edge_index[0].astype(jnp.int32)
    dst = edge_index[1].astype(jnp.int32)
    perm = jnp.argsort(dst, stable=True).astype(jnp.int32)
    src_s = src[perm]
    dst_s = dst[perm]
    zpad = jnp.zeros((CHUNK,), jnp.int32)
    srcp = jnp.concatenate([src_s, zpad])
    src2 = jnp.stack([srcp, srcp + N])
    eidp = jnp.concatenate([perm, zpad])
    eid2 = jnp.stack([eidp, eidp + E])
    seg_start = jnp.searchsorted(dst_s, dst_s, side="left").astype(jnp.int32)
    cuts = jnp.asarray(SHARD_CUTS, jnp.int32)
    pos = jnp.arange(E, dtype=jnp.int32)
    kle = jnp.searchsorted(cuts, pos, side="right").astype(jnp.int32) - 1
    kle0 = jnp.maximum(kle, 0)
    bval = cuts[kle0]
    valid = (kle >= 0) & (bval > seg_start)
    tile_of_edge = dst_s // ZPT
    tile_of_cut = dst_s[cuts] // ZPT
    slot_of_cut = (jnp.arange(len(SHARD_CUTS), dtype=jnp.int32)
                   - jnp.searchsorted(tile_of_cut, tile_of_cut, side="left"
                                      ).astype(jnp.int32))
    dst_mod = jnp.where(valid,
                        tile_of_edge * ZPT + ZPT + slot_of_cut[kle0],
                        dst_s)
    tbl = (jnp.full((NT, SLOTS), ZPT, jnp.int32)
           + jnp.arange(SLOTS, dtype=jnp.int32)[None, :])
    tbl = tbl.at[tile_of_cut, slot_of_cut].set(dst_s[cuts] - tile_of_cut * ZPT)

    dstp = jnp.concatenate([dst_mod,
                            jnp.full((CHUNK,), ACC_ROWS, jnp.int32)])
    edges = jnp.searchsorted(dst_s,
                             jnp.arange(NT + 1, dtype=jnp.int32) * ZPT
                             ).astype(jnp.int32)
    bnd = jnp.stack([edges[:-1], edges[1:]], axis=1)

    z3 = z.astype(jnp.int32).reshape(NB, BR, 1)
    ch3 = chirality.reshape(NB, BR, 1)
    fc3 = formal_charge.reshape(NB, BR, 1)
    row = lambda v: v.reshape(1, -1)

    h_split = pl.pallas_call(
        _node_prologue_body,
        grid=(NB,),
        in_specs=[
            pl.BlockSpec((1, BR, 1), lambda i: (i, 0, 0)),
            pl.BlockSpec((1, BR, 1), lambda i: (i, 0, 0)),
            pl.BlockSpec((1, BR, 1), lambda i: (i, 0, 0)),
            _full(), _full(), _full(), _full(), _full(),
        ],
        out_specs=pl.BlockSpec((NSC, BR, HP), lambda i: (0, i, 0)),
        out_shape=jax.ShapeDtypeStruct((NSC, N, HP), F32),
    )(z3, ch3, fc3, atom_table, nap_W1, row(nap_b1), nap_W2, row(nap_b2))

    EB = 2000
    emb_split = pl.pallas_call(
        _edge_prologue_body,
        grid=(E // EB,),
        in_specs=[
            pl.BlockSpec((EB, 3), lambda i: (i, 0)),
            _full(), _full(), _full(), _full(),
        ],
        out_specs=pl.BlockSpec((NSC, EB, HP), lambda i: (0, i, 0)),
        out_shape=jax.ShapeDtypeStruct((NSC, E, HP), F32),
    )(edge_attr, ee_W1, row(ee_b1), ee_W2, row(ee_b2))

    h = h_split.reshape(NSC * N, HP)

    mlp_call = pl.pallas_call(
        _mlp_body,
        grid=(NB,),
        in_specs=[
            pl.BlockSpec((NSC, BR, HP), lambda i: (0, i, 0)),
            pl.BlockSpec((NSC, BR, HP), lambda i: (0, i, 0)),
            _full(), _full(), _full(), _full(),
        ],
        out_specs=[
            pl.BlockSpec((BR, D), lambda i: (i, 0)),
            pl.BlockSpec((8, D), lambda i: (0, 0)),
        ],
        out_shape=[
            jax.ShapeDtypeStruct((N, D), F32),
            jax.ShapeDtypeStruct((8, D), F32),
        ],
    )

    bn_call = pl.pallas_call(
        _bn_body,
        grid=(NB,),
        in_specs=[
            pl.BlockSpec((BR, D), lambda i: (i, 0)),
            _full(), _full(), _full(),
        ],
        out_specs=pl.BlockSpec((NSC, BR, HP), lambda i: (0, i, 0)),
        out_shape=jax.ShapeDtypeStruct((NSC, N, HP), F32),
    )

    emb_flat = emb_split.reshape(NSC * E, HP)
    sc_aggr = _get_sc_aggr()
    for i in range(L):
        aggr = sc_aggr(h, emb_flat, src2, eid2, dstp, bnd, tbl)
        t, st = mlp_call(h.reshape(NSC, N, HP), aggr,
                         mlp_W1[i], row(mlp_b1[i]), mlp_W2[i],
                         row(mlp_b2[i]))
        h_split = bn_call(t, st, row(bn_gamma[i]), row(bn_beta[i]))
        h = h_split.reshape(NSC * N, HP)

    logits, dists = pl.pallas_call(
        _head_body,
        grid=(NB,),
        in_specs=[
            pl.BlockSpec((NSC, BR, HP), lambda i: (0, i, 0)),
            _full(), _full(), _full(), _full(),
        ],
        out_specs=[
            pl.BlockSpec((BR, NUM_EMB), lambda i: (i, 0)),
            pl.BlockSpec((BR, 6), lambda i: (i, 0)),
        ],
        out_shape=[
            jax.ShapeDtypeStruct((N, NUM_EMB), F32),
            jax.ShapeDtypeStruct((N, 6), F32),
        ],
    )(h_split, head_W, row(head_b), coord_W, row(coord_b))

    return (logits, dists)
